# baseline (speedup 1.0000x reference)
import functools

import jax
import jax.numpy as jnp
from jax import lax
from jax.experimental import pallas as pl
from jax.experimental.pallas import tpu as pltpu
from jax.experimental.pallas import tpu_sc as plsc

N_NODES = 10000
N_EDGES = 320000
N_GRAPHS = 64
D = 128
D_EXTRA = 16

NC = 2
NS = 16
LANES = 16

_HIGH = lax.Precision.HIGHEST


def _mesh():
    return plsc.VectorSubcoreMesh(core_axis_name="c", subcore_axis_name="s",
                                  num_cores=NC, num_subcores=NS)


def _sc_degrees(edge_flat):
    E_PER_SUB = N_EDGES // NS
    CH = 128
    NFULL = E_PER_SUB // CH
    REM = E_PER_SUB - NFULL * CH
    STRIPE = 624
    TAIL = N_NODES - NS * STRIPE

    DEPTH = 6
    NSET = 2
    GRP = DEPTH * NSET
    NITER = NFULL // GRP

    @functools.partial(
        pl.kernel,
        out_type=jax.ShapeDtypeStruct((2, N_NODES, D), jnp.float32),
        mesh=_mesh(),
        scratch_types=[
            [[pltpu.VMEM((CH,), jnp.int32) for _ in range(DEPTH)]
             for _ in range(NSET)],
            pltpu.VMEM((REM,), jnp.int32),
            pltpu.VMEM((CH, D), jnp.float32),
            pltpu.VMEM_SHARED((N_NODES, D), jnp.float32),
            [pltpu.SemaphoreType.DMA((DEPTH,)) for _ in range(NSET)],
            [pltpu.SemaphoreType.DMA((DEPTH,)) for _ in range(NSET)],
        ],
    )
    def k(ei_hbm, out_hbm, idx_vs, idxr_v, ones_v, acc_sh, sem_i, sem_s):
        cid = lax.axis_index("c")
        sid = lax.axis_index("s")

        @pl.loop(0, CH)
        def _(i):
            for j in range(D // LANES):
                ones_v[i, pl.ds(j * LANES, LANES)] = jnp.zeros(
                    (LANES,), jnp.float32)

        for t in range(5):
            off = t * CH
            sz = min(CH, STRIPE - off)
            pltpu.sync_copy(ones_v.at[pl.ds(0, sz)],
                            acc_sh.at[pl.ds(sid * STRIPE + off, sz)])

        @pl.when(sid == NS - 1)
        def _():
            pltpu.sync_copy(ones_v.at[pl.ds(0, TAIL)],
                            acc_sh.at[pl.ds(NS * STRIPE, TAIL)])
        plsc.subcore_barrier()

        @pl.loop(0, CH)
        def _(i):
            for j in range(D // LANES):
                ones_v[i, pl.ds(j * LANES, LANES)] = jnp.ones(
                    (LANES,), jnp.float32)

        def scat_wait(p, u):
            pltpu.make_async_copy(ones_v, acc_sh.at[idx_vs[p][u]],
                                  sem_s[p].at[u]).wait()

        @pl.loop(0, NITER)
        def _(it):
            for p in range(NSET):
                k0 = it * GRP + p * DEPTH

                @pl.when(it > 0)
                def _():
                    for u in range(DEPTH):
                        scat_wait(p, u)

                di = []
                for u in range(DEPTH):
                    base = cid * N_EDGES + sid * E_PER_SUB + (k0 + u) * CH
                    di.append(pltpu.async_copy(
                        ei_hbm.at[pl.ds(base, CH)], idx_vs[p][u],
                        sem_i[p].at[u]))
                for u in range(DEPTH):
                    di[u].wait()
                    pltpu.async_copy(
                        ones_v, acc_sh.at[idx_vs[p][u]], sem_s[p].at[u],
                        add=True)

        for p in range(NSET):
            for u in range(DEPTH):
                scat_wait(p, u)

        base = cid * N_EDGES + sid * E_PER_SUB + NFULL * CH
        pltpu.sync_copy(ei_hbm.at[pl.ds(base, REM)], idxr_v)
        pltpu.sync_copy(ones_v.at[pl.ds(0, REM)], acc_sh.at[idxr_v], add=True)

        plsc.subcore_barrier()
        pltpu.sync_copy(acc_sh.at[pl.ds(sid * STRIPE, STRIPE)],
                        out_hbm.at[cid, pl.ds(sid * STRIPE, STRIPE)])

        @pl.when(sid == NS - 1)
        def _():
            pltpu.sync_copy(acc_sh.at[pl.ds(NS * STRIPE, TAIL)],
                            out_hbm.at[cid, pl.ds(NS * STRIPE, TAIL)])

    return k(edge_flat)


def _sc_aggregate(vals, src, dst):
    E_PER_CORE = N_EDGES // NC
    E_PER_SUB = E_PER_CORE // NS
    CH = 32
    NFULL = E_PER_SUB // CH
    REM = E_PER_SUB - NFULL * CH
    STRIPE = 624
    TAIL = N_NODES - NS * STRIPE

    DEPTH = 4
    NSET = 2
    GRP = DEPTH * NSET
    NITER = NFULL // GRP

    @functools.partial(
        pl.kernel,
        out_type=jax.ShapeDtypeStruct((NC, N_NODES, D), jnp.float32),
        mesh=_mesh(),
        scratch_types=[
            [[pltpu.VMEM((CH,), jnp.int32) for _ in range(DEPTH)]
             for _ in range(NSET)],
            [[pltpu.VMEM((CH,), jnp.int32) for _ in range(DEPTH)]
             for _ in range(NSET)],
            pltpu.VMEM((REM,), jnp.int32),
            pltpu.VMEM((REM,), jnp.int32),
            [[pltpu.VMEM((CH, D), jnp.float32) for _ in range(DEPTH)]
             for _ in range(NSET)],
            pltpu.VMEM_SHARED((N_NODES, D), jnp.float32),
            [pltpu.SemaphoreType.DMA((DEPTH,)) for _ in range(NSET)],
            [pltpu.SemaphoreType.DMA((DEPTH,)) for _ in range(NSET)],
            [pltpu.SemaphoreType.DMA((DEPTH,)) for _ in range(NSET)],
        ],
    )
    def k(vals_hbm, src_hbm, dst_hbm, out_hbm,
          src_vs, dst_vs, srcr_v, dstr_v, rows_vs, acc_sh,
          sem_i, sem_g, sem_s):
        cid = lax.axis_index("c")
        sid = lax.axis_index("s")

        zb = rows_vs[0][0]

        @pl.loop(0, CH)
        def _(i):
            for j in range(D // LANES):
                zb[i, pl.ds(j * LANES, LANES)] = jnp.zeros(
                    (LANES,), jnp.float32)

        for t in range(0, STRIPE, CH):
            sz = min(CH, STRIPE - t)
            pltpu.sync_copy(zb.at[pl.ds(0, sz)],
                            acc_sh.at[pl.ds(sid * STRIPE + t, sz)])

        @pl.when(sid == NS - 1)
        def _():
            pltpu.sync_copy(zb.at[pl.ds(0, TAIL)],
                            acc_sh.at[pl.ds(NS * STRIPE, TAIL)])
        plsc.subcore_barrier()

        def scat_wait(p, u):
            pltpu.make_async_copy(rows_vs[p][u],
                                  acc_sh.at[dst_vs[p][u]],
                                  sem_s[p].at[u]).wait()

        @pl.loop(0, NITER)
        def _(it):
            for p in range(NSET):
                k0 = it * GRP + p * DEPTH

                @pl.when(it > 0)
                def _():
                    for u in range(DEPTH):
                        scat_wait(p, u)

                di = []
                for u in range(DEPTH):
                    base = (cid * E_PER_CORE + sid * E_PER_SUB
                            + (k0 + u) * CH)
                    di.append((
                        pltpu.async_copy(src_hbm.at[pl.ds(base, CH)],
                                         src_vs[p][u], sem_i[p].at[u]),
                        pltpu.async_copy(dst_hbm.at[pl.ds(base, CH)],
                                         dst_vs[p][u], sem_i[p].at[u])))
                dg = []
                for u in range(DEPTH):
                    di[u][0].wait()
                    di[u][1].wait()
                    dg.append(pltpu.async_copy(
                        vals_hbm.at[src_vs[p][u]], rows_vs[p][u],
                        sem_g[p].at[u]))
                for u in range(DEPTH):
                    dg[u].wait()
                    pltpu.async_copy(rows_vs[p][u],
                                     acc_sh.at[dst_vs[p][u]],
                                     sem_s[p].at[u], add=True)

        for p in range(NSET):
            for u in range(DEPTH):
                scat_wait(p, u)

        base = cid * E_PER_CORE + sid * E_PER_SUB + NFULL * CH
        pltpu.sync_copy(src_hbm.at[pl.ds(base, REM)], srcr_v)
        pltpu.sync_copy(dst_hbm.at[pl.ds(base, REM)], dstr_v)
        pltpu.sync_copy(vals_hbm.at[srcr_v], zb.at[pl.ds(0, REM)])
        pltpu.sync_copy(zb.at[pl.ds(0, REM)], acc_sh.at[dstr_v], add=True)

        plsc.subcore_barrier()
        pltpu.sync_copy(acc_sh.at[pl.ds(sid * STRIPE, STRIPE)],
                        out_hbm.at[cid, pl.ds(sid * STRIPE, STRIPE)])

        @pl.when(sid == NS - 1)
        def _():
            pltpu.sync_copy(acc_sh.at[pl.ds(NS * STRIPE, TAIL)],
                            out_hbm.at[cid, pl.ds(NS * STRIPE, TAIL)])

    return k(vals, src, dst)


_BLK = 1000
_NBLK = N_NODES // _BLK


def _norm_from_hist(h_col):
    return lax.rsqrt(jnp.where(h_col > 0.0, h_col, 1.0))


def _tc_matmul(x, W1):
    def body(x_ref, w_ref, o_ref):
        o_ref[...] = jnp.dot(x_ref[...], w_ref[...], precision=_HIGH)

    return pl.pallas_call(
        body,
        grid=(_NBLK,),
        in_specs=[
            pl.BlockSpec((_BLK, D), lambda i: (i, 0)),
            pl.BlockSpec((D, D), lambda i: (0, 0)),
        ],
        out_specs=pl.BlockSpec((_BLK, D), lambda i: (i, 0)),
        out_shape=jax.ShapeDtypeStruct((N_NODES, D), jnp.float32),
    )(x, W1)


def _tc_scale(t0, hist_src):
    def body(hs_ref, t_ref, o_ref):
        ns = _norm_from_hist(hs_ref[:, 0])
        o_ref[...] = t_ref[...] * ns[:, None]

    return pl.pallas_call(
        body,
        grid=(_NBLK,),
        in_specs=[
            pl.BlockSpec((_BLK, D), lambda i: (i, 0)),
            pl.BlockSpec((_BLK, D), lambda i: (i, 0)),
        ],
        out_specs=pl.BlockSpec((_BLK, D), lambda i: (i, 0)),
        out_shape=jax.ShapeDtypeStruct((N_NODES, D), jnp.float32),
    )(hist_src, t0)


def _tc_mid(p, hist_dst, b1, W2, hist_src):
    def body(p_ref, hd_ref, hs_ref, b_ref, w_ref, o_ref):
        agg = p_ref[0] + p_ref[1]
        nd = _norm_from_hist(hd_ref[:, 0])
        h = jnp.maximum(agg * nd[:, None] + b_ref[...][None, :], 0.0)
        ns = _norm_from_hist(hs_ref[:, 0])
        o_ref[...] = jnp.dot(h, w_ref[...], precision=_HIGH) * ns[:, None]

    return pl.pallas_call(
        body,
        grid=(_NBLK,),
        in_specs=[
            pl.BlockSpec((NC, _BLK, D), lambda i: (0, i, 0)),
            pl.BlockSpec((_BLK, D), lambda i: (i, 0)),
            pl.BlockSpec((_BLK, D), lambda i: (i, 0)),
            pl.BlockSpec((D,), lambda i: (0,)),
            pl.BlockSpec((D, D), lambda i: (0, 0)),
        ],
        out_specs=pl.BlockSpec((_BLK, D), lambda i: (i, 0)),
        out_shape=jax.ShapeDtypeStruct((N_NODES, D), jnp.float32),
    )(p, hist_dst, hist_src, b1, W2)


def _tc_head(p, hist_dst, b2, gid3, descriptors,
             Wc1, bc1, Wc2, bc2, Wc3, bc3):
    DC = D + D_EXTRA

    def body(p_ref, hd_ref, b_ref, g_ref, d_ref,
             w1_ref, c1_ref, w2_ref, c2_ref, w3_ref, c3_ref,
             o_ref, sums, cnts):
        i = pl.program_id(0)

        @pl.when(i == 0)
        def _():
            sums[...] = jnp.zeros_like(sums)
            cnts[...] = jnp.zeros_like(cnts)

        agg = p_ref[0] + p_ref[1]
        nd = _norm_from_hist(hd_ref[:, 0])
        h2 = jnp.maximum(agg * nd[:, None] + b_ref[...][None, :], 0.0)
        gid = g_ref[0, 0, :]
        og = (lax.broadcasted_iota(jnp.int32, (N_GRAPHS, _BLK), 0)
              == gid[None, :]).astype(jnp.float32)
        sums[...] += jnp.dot(og, h2, precision=_HIGH)
        cnts[...] += jnp.sum(og, axis=1)

        @pl.when(i == _NBLK - 1)
        def _():
            hg = sums[...] / jnp.maximum(cnts[...], 1.0)[:, None]
            z1 = (jnp.dot(hg, w1_ref[0:D, :], precision=_HIGH)
                  + jnp.dot(d_ref[...], w1_ref[D:DC, :], precision=_HIGH)
                  + c1_ref[...][None, :])
            o1 = jnp.maximum(z1, 0.0)
            o2 = jnp.maximum(
                jnp.dot(o1, w2_ref[...], precision=_HIGH)
                + c2_ref[...][None, :], 0.0)
            o_ref[...] = (jnp.dot(o2, w3_ref[...], precision=_HIGH)
                          + c3_ref[...][None, :])

    return pl.pallas_call(
        body,
        grid=(_NBLK,),
        in_specs=[
            pl.BlockSpec((NC, _BLK, D), lambda i: (0, i, 0)),
            pl.BlockSpec((_BLK, D), lambda i: (i, 0)),
            pl.BlockSpec((D,), lambda i: (0,)),
            pl.BlockSpec((1, 1, _BLK), lambda i: (i, 0, 0)),
            pl.BlockSpec((N_GRAPHS, D_EXTRA), lambda i: (0, 0)),
            pl.BlockSpec((DC, DC), lambda i: (0, 0)),
            pl.BlockSpec((DC,), lambda i: (0,)),
            pl.BlockSpec((DC, DC), lambda i: (0, 0)),
            pl.BlockSpec((DC,), lambda i: (0,)),
            pl.BlockSpec((DC, 1), lambda i: (0, 0)),
            pl.BlockSpec((1,), lambda i: (0,)),
        ],
        out_specs=pl.BlockSpec((N_GRAPHS, 1), lambda i: (0, 0)),
        out_shape=jax.ShapeDtypeStruct((N_GRAPHS, 1), jnp.float32),
        scratch_shapes=[
            pltpu.VMEM((N_GRAPHS, D), jnp.float32),
            pltpu.VMEM((N_GRAPHS,), jnp.float32),
        ],
    )(p, hist_dst, b2, gid3, descriptors, Wc1, bc1, Wc2, bc2, Wc3, bc3)


def kernel(x, edge_index, graph_ids, descriptors,
           W1, b1, W2, b2, Wc1, bc1, Wc2, bc2, Wc3, bc3):
    src = edge_index[0]
    dst = edge_index[1]
    t0 = _tc_matmul(x, W1)
    hist = _sc_degrees(edge_index.reshape(-1))
    hs = hist[0]
    hd = hist[1]
    t1 = _tc_scale(t0, hs)
    p1 = _sc_aggregate(t1, src, dst)
    t2 = _tc_mid(p1, hd, b1, W2, hs)
    p2 = _sc_aggregate(t2, src, dst)
    gid3 = graph_ids.reshape(_NBLK, 1, _BLK)
    return _tc_head(p2, hd, b2, gid3, descriptors,
                    Wc1, bc1, Wc2, bc2, Wc3, bc3)


# speedup vs baseline: 1.0747x; 1.0747x over previous
import functools

import jax
import jax.numpy as jnp
from jax import lax
from jax.experimental import pallas as pl
from jax.experimental.pallas import tpu as pltpu
from jax.experimental.pallas import tpu_sc as plsc

N_NODES = 10000
N_EDGES = 320000
N_GRAPHS = 64
D = 128
D_EXTRA = 16

NC = 2
NS = 16
LANES = 16

_HIGH = lax.Precision.HIGHEST


def _mesh():
    return plsc.VectorSubcoreMesh(core_axis_name="c", subcore_axis_name="s",
                                  num_cores=NC, num_subcores=NS)


def _sc_degrees(edge_flat):
    E_PER_SUB = N_EDGES // NS
    CH = 128
    NFULL = E_PER_SUB // CH
    REM = E_PER_SUB - NFULL * CH
    STRIPE = 624
    TAIL = N_NODES - NS * STRIPE

    DEPTH = 6
    NSET = 2
    GRP = DEPTH * NSET
    NITER = NFULL // GRP

    @functools.partial(
        pl.kernel,
        out_type=jax.ShapeDtypeStruct((2, N_NODES, D), jnp.float32),
        mesh=_mesh(),
        scratch_types=[
            [[pltpu.VMEM((CH,), jnp.int32) for _ in range(DEPTH)]
             for _ in range(NSET)],
            pltpu.VMEM((REM,), jnp.int32),
            pltpu.VMEM((CH, D), jnp.float32),
            pltpu.VMEM_SHARED((N_NODES, D), jnp.float32),
            [pltpu.SemaphoreType.DMA((DEPTH,)) for _ in range(NSET)],
            [pltpu.SemaphoreType.DMA((DEPTH,)) for _ in range(NSET)],
        ],
    )
    def k(ei_hbm, out_hbm, idx_vs, idxr_v, ones_v, acc_sh, sem_i, sem_s):
        cid = lax.axis_index("c")
        sid = lax.axis_index("s")

        @pl.loop(0, CH)
        def _(i):
            for j in range(D // LANES):
                ones_v[i, pl.ds(j * LANES, LANES)] = jnp.zeros(
                    (LANES,), jnp.float32)

        for t in range(5):
            off = t * CH
            sz = min(CH, STRIPE - off)
            pltpu.sync_copy(ones_v.at[pl.ds(0, sz)],
                            acc_sh.at[pl.ds(sid * STRIPE + off, sz)])

        @pl.when(sid == NS - 1)
        def _():
            pltpu.sync_copy(ones_v.at[pl.ds(0, TAIL)],
                            acc_sh.at[pl.ds(NS * STRIPE, TAIL)])
        plsc.subcore_barrier()

        @pl.loop(0, CH)
        def _(i):
            for j in range(D // LANES):
                ones_v[i, pl.ds(j * LANES, LANES)] = jnp.ones(
                    (LANES,), jnp.float32)

        def scat_wait(p, u):
            pltpu.make_async_copy(ones_v, acc_sh.at[idx_vs[p][u]],
                                  sem_s[p].at[u]).wait()

        @pl.loop(0, NITER)
        def _(it):
            for p in range(NSET):
                k0 = it * GRP + p * DEPTH

                @pl.when(it > 0)
                def _():
                    for u in range(DEPTH):
                        scat_wait(p, u)

                di = []
                for u in range(DEPTH):
                    base = cid * N_EDGES + sid * E_PER_SUB + (k0 + u) * CH
                    di.append(pltpu.async_copy(
                        ei_hbm.at[pl.ds(base, CH)], idx_vs[p][u],
                        sem_i[p].at[u]))
                for u in range(DEPTH):
                    di[u].wait()
                    pltpu.async_copy(
                        ones_v, acc_sh.at[idx_vs[p][u]], sem_s[p].at[u],
                        add=True)

        for p in range(NSET):
            for u in range(DEPTH):
                scat_wait(p, u)

        base = cid * N_EDGES + sid * E_PER_SUB + NFULL * CH
        pltpu.sync_copy(ei_hbm.at[pl.ds(base, REM)], idxr_v)
        pltpu.sync_copy(ones_v.at[pl.ds(0, REM)], acc_sh.at[idxr_v], add=True)

        plsc.subcore_barrier()
        pltpu.sync_copy(acc_sh.at[pl.ds(sid * STRIPE, STRIPE)],
                        out_hbm.at[cid, pl.ds(sid * STRIPE, STRIPE)])

        @pl.when(sid == NS - 1)
        def _():
            pltpu.sync_copy(acc_sh.at[pl.ds(NS * STRIPE, TAIL)],
                            out_hbm.at[cid, pl.ds(NS * STRIPE, TAIL)])

    return k(edge_flat)


def _sc_aggregate(vals, src, dst):
    E_PER_CORE = N_EDGES // NC
    E_PER_SUB = E_PER_CORE // NS
    CH = 64
    NFULL = E_PER_SUB // CH
    REM = E_PER_SUB - NFULL * CH
    STRIPE = 624
    TAIL = N_NODES - NS * STRIPE

    DEPTH = 3
    NSET = 2
    GRP = DEPTH * NSET
    NITER = NFULL // GRP

    @functools.partial(
        pl.kernel,
        out_type=jax.ShapeDtypeStruct((NC, N_NODES, D), jnp.float32),
        mesh=_mesh(),
        scratch_types=[
            [[pltpu.VMEM((CH,), jnp.int32) for _ in range(DEPTH)]
             for _ in range(NSET)],
            [[pltpu.VMEM((CH,), jnp.int32) for _ in range(DEPTH)]
             for _ in range(NSET)],
            pltpu.VMEM((REM,), jnp.int32),
            pltpu.VMEM((REM,), jnp.int32),
            [[pltpu.VMEM((CH, D), jnp.float32) for _ in range(DEPTH)]
             for _ in range(NSET)],
            pltpu.VMEM_SHARED((N_NODES, D), jnp.float32),
            [pltpu.SemaphoreType.DMA((DEPTH,)) for _ in range(NSET)],
            [pltpu.SemaphoreType.DMA((DEPTH,)) for _ in range(NSET)],
            [pltpu.SemaphoreType.DMA((DEPTH,)) for _ in range(NSET)],
        ],
    )
    def k(vals_hbm, src_hbm, dst_hbm, out_hbm,
          src_vs, dst_vs, srcr_v, dstr_v, rows_vs, acc_sh,
          sem_i, sem_g, sem_s):
        cid = lax.axis_index("c")
        sid = lax.axis_index("s")

        zb = rows_vs[0][0]

        @pl.loop(0, CH)
        def _(i):
            for j in range(D // LANES):
                zb[i, pl.ds(j * LANES, LANES)] = jnp.zeros(
                    (LANES,), jnp.float32)

        for t in range(0, STRIPE, CH):
            sz = min(CH, STRIPE - t)
            pltpu.sync_copy(zb.at[pl.ds(0, sz)],
                            acc_sh.at[pl.ds(sid * STRIPE + t, sz)])

        @pl.when(sid == NS - 1)
        def _():
            pltpu.sync_copy(zb.at[pl.ds(0, TAIL)],
                            acc_sh.at[pl.ds(NS * STRIPE, TAIL)])
        plsc.subcore_barrier()

        def scat_wait(p, u):
            pltpu.make_async_copy(rows_vs[p][u],
                                  acc_sh.at[dst_vs[p][u]],
                                  sem_s[p].at[u]).wait()

        @pl.loop(0, NITER)
        def _(it):
            for p in range(NSET):
                k0 = it * GRP + p * DEPTH

                @pl.when(it > 0)
                def _():
                    for u in range(DEPTH):
                        scat_wait(p, u)

                di = []
                for u in range(DEPTH):
                    base = (cid * E_PER_CORE + sid * E_PER_SUB
                            + (k0 + u) * CH)
                    di.append((
                        pltpu.async_copy(src_hbm.at[pl.ds(base, CH)],
                                         src_vs[p][u], sem_i[p].at[u]),
                        pltpu.async_copy(dst_hbm.at[pl.ds(base, CH)],
                                         dst_vs[p][u], sem_i[p].at[u])))
                dg = []
                for u in range(DEPTH):
                    di[u][0].wait()
                    di[u][1].wait()
                    dg.append(pltpu.async_copy(
                        vals_hbm.at[src_vs[p][u]], rows_vs[p][u],
                        sem_g[p].at[u]))
                for u in range(DEPTH):
                    dg[u].wait()
                    pltpu.async_copy(rows_vs[p][u],
                                     acc_sh.at[dst_vs[p][u]],
                                     sem_s[p].at[u], add=True)

        for p in range(NSET):
            for u in range(DEPTH):
                scat_wait(p, u)

        base = cid * E_PER_CORE + sid * E_PER_SUB + NFULL * CH
        pltpu.sync_copy(src_hbm.at[pl.ds(base, REM)], srcr_v)
        pltpu.sync_copy(dst_hbm.at[pl.ds(base, REM)], dstr_v)
        pltpu.sync_copy(vals_hbm.at[srcr_v], zb.at[pl.ds(0, REM)])
        pltpu.sync_copy(zb.at[pl.ds(0, REM)], acc_sh.at[dstr_v], add=True)

        plsc.subcore_barrier()
        pltpu.sync_copy(acc_sh.at[pl.ds(sid * STRIPE, STRIPE)],
                        out_hbm.at[cid, pl.ds(sid * STRIPE, STRIPE)])

        @pl.when(sid == NS - 1)
        def _():
            pltpu.sync_copy(acc_sh.at[pl.ds(NS * STRIPE, TAIL)],
                            out_hbm.at[cid, pl.ds(NS * STRIPE, TAIL)])

    return k(vals, src, dst)


_BLK = 1000
_NBLK = N_NODES // _BLK


def _norm_from_hist(h_col):
    return lax.rsqrt(jnp.where(h_col > 0.0, h_col, 1.0))


def _tc_matmul(x, W1):
    def body(x_ref, w_ref, o_ref):
        o_ref[...] = jnp.dot(x_ref[...], w_ref[...], precision=_HIGH)

    return pl.pallas_call(
        body,
        grid=(_NBLK,),
        in_specs=[
            pl.BlockSpec((_BLK, D), lambda i: (i, 0)),
            pl.BlockSpec((D, D), lambda i: (0, 0)),
        ],
        out_specs=pl.BlockSpec((_BLK, D), lambda i: (i, 0)),
        out_shape=jax.ShapeDtypeStruct((N_NODES, D), jnp.float32),
    )(x, W1)


def _tc_scale(t0, hist_src):
    def body(hs_ref, t_ref, o_ref):
        ns = _norm_from_hist(hs_ref[:, 0])
        o_ref[...] = t_ref[...] * ns[:, None]

    return pl.pallas_call(
        body,
        grid=(_NBLK,),
        in_specs=[
            pl.BlockSpec((_BLK, D), lambda i: (i, 0)),
            pl.BlockSpec((_BLK, D), lambda i: (i, 0)),
        ],
        out_specs=pl.BlockSpec((_BLK, D), lambda i: (i, 0)),
        out_shape=jax.ShapeDtypeStruct((N_NODES, D), jnp.float32),
    )(hist_src, t0)


def _tc_mid(p, hist_dst, b1, W2, hist_src):
    def body(p_ref, hd_ref, hs_ref, b_ref, w_ref, o_ref):
        agg = p_ref[0] + p_ref[1]
        nd = _norm_from_hist(hd_ref[:, 0])
        h = jnp.maximum(agg * nd[:, None] + b_ref[...][None, :], 0.0)
        ns = _norm_from_hist(hs_ref[:, 0])
        o_ref[...] = jnp.dot(h, w_ref[...], precision=_HIGH) * ns[:, None]

    return pl.pallas_call(
        body,
        grid=(_NBLK,),
        in_specs=[
            pl.BlockSpec((NC, _BLK, D), lambda i: (0, i, 0)),
            pl.BlockSpec((_BLK, D), lambda i: (i, 0)),
            pl.BlockSpec((_BLK, D), lambda i: (i, 0)),
            pl.BlockSpec((D,), lambda i: (0,)),
            pl.BlockSpec((D, D), lambda i: (0, 0)),
        ],
        out_specs=pl.BlockSpec((_BLK, D), lambda i: (i, 0)),
        out_shape=jax.ShapeDtypeStruct((N_NODES, D), jnp.float32),
    )(p, hist_dst, hist_src, b1, W2)


def _tc_head(p, hist_dst, b2, gid3, descriptors,
             Wc1, bc1, Wc2, bc2, Wc3, bc3):
    DC = D + D_EXTRA

    def body(p_ref, hd_ref, b_ref, g_ref, d_ref,
             w1_ref, c1_ref, w2_ref, c2_ref, w3_ref, c3_ref,
             o_ref, sums, cnts):
        i = pl.program_id(0)

        @pl.when(i == 0)
        def _():
            sums[...] = jnp.zeros_like(sums)
            cnts[...] = jnp.zeros_like(cnts)

        agg = p_ref[0] + p_ref[1]
        nd = _norm_from_hist(hd_ref[:, 0])
        h2 = jnp.maximum(agg * nd[:, None] + b_ref[...][None, :], 0.0)
        gid = g_ref[0, 0, :]
        og = (lax.broadcasted_iota(jnp.int32, (N_GRAPHS, _BLK), 0)
              == gid[None, :]).astype(jnp.float32)
        sums[...] += jnp.dot(og, h2, precision=_HIGH)
        cnts[...] += jnp.sum(og, axis=1)

        @pl.when(i == _NBLK - 1)
        def _():
            hg = sums[...] / jnp.maximum(cnts[...], 1.0)[:, None]
            z1 = (jnp.dot(hg, w1_ref[0:D, :], precision=_HIGH)
                  + jnp.dot(d_ref[...], w1_ref[D:DC, :], precision=_HIGH)
                  + c1_ref[...][None, :])
            o1 = jnp.maximum(z1, 0.0)
            o2 = jnp.maximum(
                jnp.dot(o1, w2_ref[...], precision=_HIGH)
                + c2_ref[...][None, :], 0.0)
            o_ref[...] = (jnp.dot(o2, w3_ref[...], precision=_HIGH)
                          + c3_ref[...][None, :])

    return pl.pallas_call(
        body,
        grid=(_NBLK,),
        in_specs=[
            pl.BlockSpec((NC, _BLK, D), lambda i: (0, i, 0)),
            pl.BlockSpec((_BLK, D), lambda i: (i, 0)),
            pl.BlockSpec((D,), lambda i: (0,)),
            pl.BlockSpec((1, 1, _BLK), lambda i: (i, 0, 0)),
            pl.BlockSpec((N_GRAPHS, D_EXTRA), lambda i: (0, 0)),
            pl.BlockSpec((DC, DC), lambda i: (0, 0)),
            pl.BlockSpec((DC,), lambda i: (0,)),
            pl.BlockSpec((DC, DC), lambda i: (0, 0)),
            pl.BlockSpec((DC,), lambda i: (0,)),
            pl.BlockSpec((DC, 1), lambda i: (0, 0)),
            pl.BlockSpec((1,), lambda i: (0,)),
        ],
        out_specs=pl.BlockSpec((N_GRAPHS, 1), lambda i: (0, 0)),
        out_shape=jax.ShapeDtypeStruct((N_GRAPHS, 1), jnp.float32),
        scratch_shapes=[
            pltpu.VMEM((N_GRAPHS, D), jnp.float32),
            pltpu.VMEM((N_GRAPHS,), jnp.float32),
        ],
    )(p, hist_dst, b2, gid3, descriptors, Wc1, bc1, Wc2, bc2, Wc3, bc3)


def kernel(x, edge_index, graph_ids, descriptors,
           W1, b1, W2, b2, Wc1, bc1, Wc2, bc2, Wc3, bc3):
    src = edge_index[0]
    dst = edge_index[1]
    t0 = _tc_matmul(x, W1)
    hist = _sc_degrees(edge_index.reshape(-1))
    hs = hist[0]
    hd = hist[1]
    t1 = _tc_scale(t0, hs)
    p1 = _sc_aggregate(t1, src, dst)
    t2 = _tc_mid(p1, hd, b1, W2, hs)
    p2 = _sc_aggregate(t2, src, dst)
    gid3 = graph_ids.reshape(_NBLK, 1, _BLK)
    return _tc_head(p2, hd, b2, gid3, descriptors,
                    Wc1, bc1, Wc2, bc2, Wc3, bc3)


# speedup vs baseline: 1.0903x; 1.0145x over previous
import functools

import jax
import jax.numpy as jnp
from jax import lax
from jax.experimental import pallas as pl
from jax.experimental.pallas import tpu as pltpu
from jax.experimental.pallas import tpu_sc as plsc

N_NODES = 10000
N_EDGES = 320000
N_GRAPHS = 64
D = 128
D_EXTRA = 16

NC = 2
NS = 16
LANES = 16

_HIGH = lax.Precision.HIGHEST


def _mesh():
    return plsc.VectorSubcoreMesh(core_axis_name="c", subcore_axis_name="s",
                                  num_cores=NC, num_subcores=NS)


def _sc_degrees(edge_flat):
    E_PER_SUB = N_EDGES // NS
    CH = 128
    NFULL = E_PER_SUB // CH
    REM = E_PER_SUB - NFULL * CH
    STRIPE = 624
    TAIL = N_NODES - NS * STRIPE

    DEPTH = 6
    NSET = 2
    GRP = DEPTH * NSET
    NITER = NFULL // GRP

    @functools.partial(
        pl.kernel,
        out_type=jax.ShapeDtypeStruct((2, N_NODES, D), jnp.float32),
        mesh=_mesh(),
        scratch_types=[
            [[pltpu.VMEM((CH,), jnp.int32) for _ in range(DEPTH)]
             for _ in range(NSET)],
            pltpu.VMEM((REM,), jnp.int32),
            pltpu.VMEM((CH, D), jnp.float32),
            pltpu.VMEM_SHARED((N_NODES, D), jnp.float32),
            [pltpu.SemaphoreType.DMA((DEPTH,)) for _ in range(NSET)],
            [pltpu.SemaphoreType.DMA((DEPTH,)) for _ in range(NSET)],
        ],
    )
    def k(ei_hbm, out_hbm, idx_vs, idxr_v, ones_v, acc_sh, sem_i, sem_s):
        cid = lax.axis_index("c")
        sid = lax.axis_index("s")

        @pl.loop(0, CH)
        def _(i):
            for j in range(D // LANES):
                ones_v[i, pl.ds(j * LANES, LANES)] = jnp.zeros(
                    (LANES,), jnp.float32)

        for t in range(5):
            off = t * CH
            sz = min(CH, STRIPE - off)
            pltpu.sync_copy(ones_v.at[pl.ds(0, sz)],
                            acc_sh.at[pl.ds(sid * STRIPE + off, sz)])

        @pl.when(sid == NS - 1)
        def _():
            pltpu.sync_copy(ones_v.at[pl.ds(0, TAIL)],
                            acc_sh.at[pl.ds(NS * STRIPE, TAIL)])
        plsc.subcore_barrier()

        @pl.loop(0, CH)
        def _(i):
            for j in range(D // LANES):
                ones_v[i, pl.ds(j * LANES, LANES)] = jnp.ones(
                    (LANES,), jnp.float32)

        def scat_wait(p, u):
            pltpu.make_async_copy(ones_v, acc_sh.at[idx_vs[p][u]],
                                  sem_s[p].at[u]).wait()

        @pl.loop(0, NITER)
        def _(it):
            for p in range(NSET):
                k0 = it * GRP + p * DEPTH

                @pl.when(it > 0)
                def _():
                    for u in range(DEPTH):
                        scat_wait(p, u)

                di = []
                for u in range(DEPTH):
                    base = cid * N_EDGES + sid * E_PER_SUB + (k0 + u) * CH
                    di.append(pltpu.async_copy(
                        ei_hbm.at[pl.ds(base, CH)], idx_vs[p][u],
                        sem_i[p].at[u]))
                for u in range(DEPTH):
                    di[u].wait()
                    pltpu.async_copy(
                        ones_v, acc_sh.at[idx_vs[p][u]], sem_s[p].at[u],
                        add=True)

        for p in range(NSET):
            for u in range(DEPTH):
                scat_wait(p, u)

        base = cid * N_EDGES + sid * E_PER_SUB + NFULL * CH
        pltpu.sync_copy(ei_hbm.at[pl.ds(base, REM)], idxr_v)
        pltpu.sync_copy(ones_v.at[pl.ds(0, REM)], acc_sh.at[idxr_v], add=True)

        plsc.subcore_barrier()
        pltpu.sync_copy(acc_sh.at[pl.ds(sid * STRIPE, STRIPE)],
                        out_hbm.at[cid, pl.ds(sid * STRIPE, STRIPE)])

        @pl.when(sid == NS - 1)
        def _():
            pltpu.sync_copy(acc_sh.at[pl.ds(NS * STRIPE, TAIL)],
                            out_hbm.at[cid, pl.ds(NS * STRIPE, TAIL)])

    return k(edge_flat)


def _sc_aggregate(vals, edge_flat):
    E_PER_CORE = N_EDGES // NC
    E_PER_SUB = E_PER_CORE // NS
    CH = 64
    NFULL = E_PER_SUB // CH
    REM = E_PER_SUB - NFULL * CH
    STRIPE = 624
    TAIL = N_NODES - NS * STRIPE

    DEPTH = 3
    NSET = 2
    GRP = DEPTH * NSET
    NITER = NFULL // GRP

    @functools.partial(
        pl.kernel,
        out_type=jax.ShapeDtypeStruct((NC, N_NODES, D), jnp.float32),
        mesh=_mesh(),
        scratch_types=[
            [[pltpu.VMEM((CH,), jnp.int32) for _ in range(DEPTH)]
             for _ in range(NSET)],
            [[pltpu.VMEM((CH,), jnp.int32) for _ in range(DEPTH)]
             for _ in range(NSET)],
            pltpu.VMEM((REM,), jnp.int32),
            pltpu.VMEM((REM,), jnp.int32),
            [[pltpu.VMEM((CH, D), jnp.float32) for _ in range(DEPTH)]
             for _ in range(NSET)],
            pltpu.VMEM_SHARED((N_NODES, D), jnp.float32),
            [pltpu.SemaphoreType.DMA((DEPTH,)) for _ in range(NSET)],
            [pltpu.SemaphoreType.DMA((DEPTH,)) for _ in range(NSET)],
            [pltpu.SemaphoreType.DMA((DEPTH,)) for _ in range(NSET)],
        ],
    )
    def k(vals_hbm, ei_hbm, out_hbm,
          src_vs, dst_vs, srcr_v, dstr_v, rows_vs, acc_sh,
          sem_i, sem_g, sem_s):
        cid = lax.axis_index("c")
        sid = lax.axis_index("s")

        zb = rows_vs[0][0]

        @pl.loop(0, CH)
        def _(i):
            for j in range(D // LANES):
                zb[i, pl.ds(j * LANES, LANES)] = jnp.zeros(
                    (LANES,), jnp.float32)

        for t in range(0, STRIPE, CH):
            sz = min(CH, STRIPE - t)
            pltpu.sync_copy(zb.at[pl.ds(0, sz)],
                            acc_sh.at[pl.ds(sid * STRIPE + t, sz)])

        @pl.when(sid == NS - 1)
        def _():
            pltpu.sync_copy(zb.at[pl.ds(0, TAIL)],
                            acc_sh.at[pl.ds(NS * STRIPE, TAIL)])
        plsc.subcore_barrier()

        def scat_wait(p, u):
            pltpu.make_async_copy(rows_vs[p][u],
                                  acc_sh.at[dst_vs[p][u]],
                                  sem_s[p].at[u]).wait()

        @pl.loop(0, NITER)
        def _(it):
            for p in range(NSET):
                k0 = it * GRP + p * DEPTH

                @pl.when(it > 0)
                def _():
                    for u in range(DEPTH):
                        scat_wait(p, u)

                di = []
                for u in range(DEPTH):
                    base = (cid * E_PER_CORE + sid * E_PER_SUB
                            + (k0 + u) * CH)
                    di.append((
                        pltpu.async_copy(ei_hbm.at[pl.ds(base, CH)],
                                         src_vs[p][u], sem_i[p].at[u]),
                        pltpu.async_copy(ei_hbm.at[pl.ds(N_EDGES + base, CH)],
                                         dst_vs[p][u], sem_i[p].at[u])))
                dg = []
                for u in range(DEPTH):
                    di[u][0].wait()
                    di[u][1].wait()
                    dg.append(pltpu.async_copy(
                        vals_hbm.at[src_vs[p][u]], rows_vs[p][u],
                        sem_g[p].at[u]))
                for u in range(DEPTH):
                    dg[u].wait()
                    pltpu.async_copy(rows_vs[p][u],
                                     acc_sh.at[dst_vs[p][u]],
                                     sem_s[p].at[u], add=True)

        for p in range(NSET):
            for u in range(DEPTH):
                scat_wait(p, u)

        base = cid * E_PER_CORE + sid * E_PER_SUB + NFULL * CH
        pltpu.sync_copy(ei_hbm.at[pl.ds(base, REM)], srcr_v)
        pltpu.sync_copy(ei_hbm.at[pl.ds(N_EDGES + base, REM)], dstr_v)
        pltpu.sync_copy(vals_hbm.at[srcr_v], zb.at[pl.ds(0, REM)])
        pltpu.sync_copy(zb.at[pl.ds(0, REM)], acc_sh.at[dstr_v], add=True)

        plsc.subcore_barrier()
        pltpu.sync_copy(acc_sh.at[pl.ds(sid * STRIPE, STRIPE)],
                        out_hbm.at[cid, pl.ds(sid * STRIPE, STRIPE)])

        @pl.when(sid == NS - 1)
        def _():
            pltpu.sync_copy(acc_sh.at[pl.ds(NS * STRIPE, TAIL)],
                            out_hbm.at[cid, pl.ds(NS * STRIPE, TAIL)])

    return k(vals, edge_flat)


_BLK = 1000
_NBLK = N_NODES // _BLK


def _norm_from_hist(h_col):
    return lax.rsqrt(jnp.where(h_col > 0.0, h_col, 1.0))


def _tc_matmul(x, W1):
    def body(x_ref, w_ref, o_ref):
        o_ref[...] = jnp.dot(x_ref[...], w_ref[...], precision=_HIGH)

    return pl.pallas_call(
        body,
        grid=(_NBLK,),
        in_specs=[
            pl.BlockSpec((_BLK, D), lambda i: (i, 0)),
            pl.BlockSpec((D, D), lambda i: (0, 0)),
        ],
        out_specs=pl.BlockSpec((_BLK, D), lambda i: (i, 0)),
        out_shape=jax.ShapeDtypeStruct((N_NODES, D), jnp.float32),
    )(x, W1)


def _tc_scale(t0, hist):
    def body(hs_ref, t_ref, o_ref):
        ns = _norm_from_hist(hs_ref[0, :, 0])
        o_ref[...] = t_ref[...] * ns[:, None]

    return pl.pallas_call(
        body,
        grid=(_NBLK,),
        in_specs=[
            pl.BlockSpec((1, _BLK, D), lambda i: (0, i, 0)),
            pl.BlockSpec((_BLK, D), lambda i: (i, 0)),
        ],
        out_specs=pl.BlockSpec((_BLK, D), lambda i: (i, 0)),
        out_shape=jax.ShapeDtypeStruct((N_NODES, D), jnp.float32),
    )(hist, t0)


def _tc_mid(p, hist, b1, W2):
    def body(p_ref, h_ref, b_ref, w_ref, o_ref):
        agg = p_ref[0] + p_ref[1]
        nd = _norm_from_hist(h_ref[1, :, 0])
        h = jnp.maximum(agg * nd[:, None] + b_ref[...][None, :], 0.0)
        ns = _norm_from_hist(h_ref[0, :, 0])
        o_ref[...] = jnp.dot(h, w_ref[...], precision=_HIGH) * ns[:, None]

    return pl.pallas_call(
        body,
        grid=(_NBLK,),
        in_specs=[
            pl.BlockSpec((NC, _BLK, D), lambda i: (0, i, 0)),
            pl.BlockSpec((2, _BLK, D), lambda i: (0, i, 0)),
            pl.BlockSpec((D,), lambda i: (0,)),
            pl.BlockSpec((D, D), lambda i: (0, 0)),
        ],
        out_specs=pl.BlockSpec((_BLK, D), lambda i: (i, 0)),
        out_shape=jax.ShapeDtypeStruct((N_NODES, D), jnp.float32),
    )(p, hist, b1, W2)


def _tc_head(p, hist, b2, gid3, descriptors,
             Wc1, bc1, Wc2, bc2, Wc3, bc3):
    DC = D + D_EXTRA

    def body(p_ref, h_ref, b_ref, g_ref, d_ref,
             w1_ref, c1_ref, w2_ref, c2_ref, w3_ref, c3_ref,
             o_ref, sums, cnts):
        i = pl.program_id(0)

        @pl.when(i == 0)
        def _():
            sums[...] = jnp.zeros_like(sums)
            cnts[...] = jnp.zeros_like(cnts)

        agg = p_ref[0] + p_ref[1]
        nd = _norm_from_hist(h_ref[1, :, 0])
        h2 = jnp.maximum(agg * nd[:, None] + b_ref[...][None, :], 0.0)
        gid = g_ref[0, 0, :]
        og = (lax.broadcasted_iota(jnp.int32, (N_GRAPHS, _BLK), 0)
              == gid[None, :]).astype(jnp.float32)
        sums[...] += jnp.dot(og, h2, precision=_HIGH)
        cnts[...] += jnp.sum(og, axis=1)

        @pl.when(i == _NBLK - 1)
        def _():
            hg = sums[...] / jnp.maximum(cnts[...], 1.0)[:, None]
            z1 = (jnp.dot(hg, w1_ref[0:D, :], precision=_HIGH)
                  + jnp.dot(d_ref[...], w1_ref[D:DC, :], precision=_HIGH)
                  + c1_ref[...][None, :])
            o1 = jnp.maximum(z1, 0.0)
            o2 = jnp.maximum(
                jnp.dot(o1, w2_ref[...], precision=_HIGH)
                + c2_ref[...][None, :], 0.0)
            o_ref[...] = (jnp.dot(o2, w3_ref[...], precision=_HIGH)
                          + c3_ref[...][None, :])

    return pl.pallas_call(
        body,
        grid=(_NBLK,),
        in_specs=[
            pl.BlockSpec((NC, _BLK, D), lambda i: (0, i, 0)),
            pl.BlockSpec((2, _BLK, D), lambda i: (0, i, 0)),
            pl.BlockSpec((D,), lambda i: (0,)),
            pl.BlockSpec((1, 1, _BLK), lambda i: (i, 0, 0)),
            pl.BlockSpec((N_GRAPHS, D_EXTRA), lambda i: (0, 0)),
            pl.BlockSpec((DC, DC), lambda i: (0, 0)),
            pl.BlockSpec((DC,), lambda i: (0,)),
            pl.BlockSpec((DC, DC), lambda i: (0, 0)),
            pl.BlockSpec((DC,), lambda i: (0,)),
            pl.BlockSpec((DC, 1), lambda i: (0, 0)),
            pl.BlockSpec((1,), lambda i: (0,)),
        ],
        out_specs=pl.BlockSpec((N_GRAPHS, 1), lambda i: (0, 0)),
        out_shape=jax.ShapeDtypeStruct((N_GRAPHS, 1), jnp.float32),
        scratch_shapes=[
            pltpu.VMEM((N_GRAPHS, D), jnp.float32),
            pltpu.VMEM((N_GRAPHS,), jnp.float32),
        ],
    )(p, hist, b2, gid3, descriptors, Wc1, bc1, Wc2, bc2, Wc3, bc3)


def kernel(x, edge_index, graph_ids, descriptors,
           W1, b1, W2, b2, Wc1, bc1, Wc2, bc2, Wc3, bc3):
    edge_flat = edge_index.reshape(-1)
    t0 = _tc_matmul(x, W1)
    hist = _sc_degrees(edge_flat)
    t1 = _tc_scale(t0, hist)
    p1 = _sc_aggregate(t1, edge_flat)
    t2 = _tc_mid(p1, hist, b1, W2)
    p2 = _sc_aggregate(t2, edge_flat)
    gid3 = graph_ids.reshape(_NBLK, 1, _BLK)
    return _tc_head(p2, hist, b2, gid3, descriptors,
                    Wc1, bc1, Wc2, bc2, Wc3, bc3)
